# Initial kernel scaffold; baseline (speedup 1.0000x reference)
import jax
import jax.numpy as jnp
from jax.experimental import pallas as pl


def kernel(inputs, neg_indices):
    raise NotImplementedError("write your pallas kernel here")


# speedup vs baseline: 1.3554x; 1.3554x over previous
import jax
import jax.numpy as jnp
from jax import lax
from jax.experimental import pallas as pl
from jax.experimental.pallas import tpu as pltpu, tpu_sc as plsc
import functools

_T_SKIP = 4
_PRED_STEPS = 12
_NUM_NEG = 10
_B, _T, _D = 4, 512, 128
_TARGET_LEN = _T - _T_SKIP - _PRED_STEPS - 1
_PER_BATCH = (_NUM_NEG + 1) * _TARGET_LEN * _PRED_STEPS
_N = _B * _PER_BATCH

_NC, _NS = 2, 16
_NW = _NC * _NS
_R = 8192
_C = 256
_NCHUNK = _R // _C


def _gather_body(x_hbm, gidx_hbm, out_hbm, idx_v, rows0, rows1, g0, g1, s0, s1):
    wid = lax.axis_index("s") * _NC + lax.axis_index("c")
    base = jnp.minimum(wid * _R, _N - _R)
    pltpu.sync_copy(gidx_hbm.at[pl.ds(base, _R)], idx_v)

    rows = (rows0, rows1)
    gsem = (g0, g1)
    ssem = (s0, s1)
    gcp = [None] * _NCHUNK
    scp = [None] * _NCHUNK
    for j in range(_NCHUNK):
        p = j & 1
        if j >= 2:
            scp[j - 2].wait()
        gcp[j] = pltpu.async_copy(
            x_hbm.at[idx_v.at[pl.ds(j * _C, _C)]], rows[p], gsem[p]
        )
        if j >= 1:
            gcp[j - 1].wait()
            scp[j - 1] = pltpu.async_copy(
                rows[1 - p], out_hbm.at[pl.ds(base + (j - 1) * _C, _C)], ssem[1 - p]
            )
    last = _NCHUNK - 1
    gcp[last].wait()
    scp[last] = pltpu.async_copy(
        rows[last & 1], out_hbm.at[pl.ds(base + last * _C, _C)], ssem[last & 1]
    )
    scp[last - 1].wait()
    scp[last].wait()


@functools.partial(jax.jit)
def _sc_gather(x_flat, g_idx):
    mesh = plsc.VectorSubcoreMesh(
        core_axis_name="c", subcore_axis_name="s", num_cores=_NC, num_subcores=_NS
    )
    return pl.kernel(
        _gather_body,
        out_type=jax.ShapeDtypeStruct((_N, _D), jnp.float32),
        mesh=mesh,
        scratch_types=[
            pltpu.VMEM((_R,), jnp.int32),
            pltpu.VMEM((_C, _D), jnp.float32),
            pltpu.VMEM((_C, _D), jnp.float32),
            pltpu.SemaphoreType.DMA,
            pltpu.SemaphoreType.DMA,
            pltpu.SemaphoreType.DMA,
            pltpu.SemaphoreType.DMA,
        ],
    )(x_flat, g_idx)


def kernel(inputs, neg_indices):
    x_flat = inputs.reshape(_B * _T, _D)
    l = jnp.arange(_TARGET_LEN, dtype=jnp.int32)
    p = jnp.arange(_PRED_STEPS, dtype=jnp.int32)
    pos = (_T_SKIP + 1 + l[:, None] + p[None, :]).reshape(-1)
    full = jnp.concatenate([pos, neg_indices])
    g_idx = (
        jnp.arange(_B, dtype=jnp.int32)[:, None] * _T + full[None, :]
    ).reshape(-1)
    out = _sc_gather(x_flat, g_idx)
    return out.reshape(_B, _NUM_NEG + 1, _TARGET_LEN, _PRED_STEPS, _D)


# speedup vs baseline: 1.3616x; 1.0046x over previous
import jax
import jax.numpy as jnp
from jax import lax
from jax.experimental import pallas as pl
from jax.experimental.pallas import tpu as pltpu, tpu_sc as plsc
import functools

_T_SKIP = 4
_PRED_STEPS = 12
_NUM_NEG = 10
_B, _T, _D = 4, 512, 128
_TARGET_LEN = _T - _T_SKIP - _PRED_STEPS - 1
_PER_BATCH = (_NUM_NEG + 1) * _TARGET_LEN * _PRED_STEPS
_N = _B * _PER_BATCH

_NC, _NS = 2, 16
_NW = _NC * _NS
_R = 8192
_C = 256
_NCHUNK = _R // _C
_NBUF = 3


def _gather_body(x_hbm, gidx_hbm, out_hbm, idx_v, *bufs_and_sems):
    rows = bufs_and_sems[:_NBUF]
    gsem = bufs_and_sems[_NBUF:2 * _NBUF]
    ssem = bufs_and_sems[2 * _NBUF:]
    wid = lax.axis_index("s") * _NC + lax.axis_index("c")
    base = jnp.minimum(wid * _R, _N - _R)
    pltpu.sync_copy(gidx_hbm.at[pl.ds(base, _R)], idx_v)

    gcp = [None] * _NCHUNK
    scp = [None] * _NCHUNK
    for j in range(_NCHUNK):
        p = j % _NBUF
        if j >= _NBUF:
            scp[j - _NBUF].wait()
        gcp[j] = pltpu.async_copy(
            x_hbm.at[idx_v.at[pl.ds(j * _C, _C)]], rows[p], gsem[p]
        )
        if j >= 1:
            gcp[j - 1].wait()
            scp[j - 1] = pltpu.async_copy(
                rows[(j - 1) % _NBUF],
                out_hbm.at[pl.ds(base + (j - 1) * _C, _C)],
                ssem[(j - 1) % _NBUF],
            )
    last = _NCHUNK - 1
    gcp[last].wait()
    scp[last] = pltpu.async_copy(
        rows[last % _NBUF], out_hbm.at[pl.ds(base + last * _C, _C)], ssem[last % _NBUF]
    )
    for j in range(max(0, _NCHUNK - _NBUF), _NCHUNK):
        scp[j].wait()


@functools.partial(jax.jit)
def _sc_gather(x_flat, g_idx):
    mesh = plsc.VectorSubcoreMesh(
        core_axis_name="c", subcore_axis_name="s", num_cores=_NC, num_subcores=_NS
    )
    return pl.kernel(
        _gather_body,
        out_type=jax.ShapeDtypeStruct((_N, _D), jnp.float32),
        mesh=mesh,
        scratch_types=(
            [pltpu.VMEM((_R,), jnp.int32)]
            + [pltpu.VMEM((_C, _D), jnp.float32)] * _NBUF
            + [pltpu.SemaphoreType.DMA] * (2 * _NBUF)
        ),
    )(x_flat, g_idx)


def kernel(inputs, neg_indices):
    x_flat = inputs.reshape(_B * _T, _D)
    l = jnp.arange(_TARGET_LEN, dtype=jnp.int32)
    p = jnp.arange(_PRED_STEPS, dtype=jnp.int32)
    pos = (_T_SKIP + 1 + l[:, None] + p[None, :]).reshape(-1)
    full = jnp.concatenate([pos, neg_indices])
    g_idx = (
        jnp.arange(_B, dtype=jnp.int32)[:, None] * _T + full[None, :]
    ).reshape(-1)
    out = _sc_gather(x_flat, g_idx)
    return out.reshape(_B, _NUM_NEG + 1, _TARGET_LEN, _PRED_STEPS, _D)


# speedup vs baseline: 1.6823x; 1.2355x over previous
import jax
import jax.numpy as jnp
from jax import lax
from jax.experimental import pallas as pl
from jax.experimental.pallas import tpu as pltpu, tpu_sc as plsc
import functools

_T_SKIP = 4
_PRED_STEPS = 12
_PSTEP_PAD = 16
_NUM_NEG = 10
_B, _T, _D = 4, 512, 128
_TARGET_LEN = _T - _T_SKIP - _PRED_STEPS - 1
_GROUPS = _B * (_NUM_NEG + 1) * _TARGET_LEN
_N = _GROUPS * _PSTEP_PAD

_NC, _NS = 2, 16
_NW = _NC * _NS
_R = 10896
_C = 256
_NBUF = 3
_CHUNKS = [(j * _C, min(_C, _R - j * _C)) for j in range((_R + _C - 1) // _C)]


def _gather_body(x_hbm, gidx_hbm, out_hbm, idx_v, *bufs_and_sems):
    rows = bufs_and_sems[:_NBUF]
    gsem = bufs_and_sems[_NBUF:2 * _NBUF]
    ssem = bufs_and_sems[2 * _NBUF:]
    wid = lax.axis_index("s") * _NC + lax.axis_index("c")
    base = jnp.minimum(wid * _R, _N - _R)
    pltpu.sync_copy(gidx_hbm.at[pl.ds(base, _R)], idx_v)

    nchunk = len(_CHUNKS)
    gcp = [None] * nchunk
    scp = [None] * nchunk
    for j, (off, sz) in enumerate(_CHUNKS):
        p = j % _NBUF
        if j >= _NBUF:
            scp[j - _NBUF].wait()
        gcp[j] = pltpu.async_copy(
            x_hbm.at[idx_v.at[pl.ds(off, sz)]],
            rows[p].at[pl.ds(0, sz)],
            gsem[p],
        )
        if j >= 1:
            po, (poff, psz) = (j - 1) % _NBUF, _CHUNKS[j - 1]
            gcp[j - 1].wait()
            scp[j - 1] = pltpu.async_copy(
                rows[po].at[pl.ds(0, psz)],
                out_hbm.at[pl.ds(base + poff, psz)],
                ssem[po],
            )
    last = nchunk - 1
    po, (poff, psz) = last % _NBUF, _CHUNKS[last]
    gcp[last].wait()
    scp[last] = pltpu.async_copy(
        rows[po].at[pl.ds(0, psz)], out_hbm.at[pl.ds(base + poff, psz)], ssem[po]
    )
    for j in range(max(0, nchunk - _NBUF), nchunk):
        scp[j].wait()


@functools.partial(jax.jit)
def _sc_gather(x_flat, g_idx):
    mesh = plsc.VectorSubcoreMesh(
        core_axis_name="c", subcore_axis_name="s", num_cores=_NC, num_subcores=_NS
    )
    return pl.kernel(
        _gather_body,
        out_type=jax.ShapeDtypeStruct((_N, _D), jnp.float32),
        mesh=mesh,
        scratch_types=(
            [pltpu.VMEM((_R,), jnp.int32)]
            + [pltpu.VMEM((_C, _D), jnp.float32)] * _NBUF
            + [pltpu.SemaphoreType.DMA] * (2 * _NBUF)
        ),
    )(x_flat, g_idx)


def kernel(inputs, neg_indices):
    x_flat = inputs.reshape(_B * _T, _D)
    l = jnp.arange(_TARGET_LEN, dtype=jnp.int32)
    p = jnp.arange(_PRED_STEPS, dtype=jnp.int32)
    pos = (_T_SKIP + 1 + l[:, None] + p[None, :]).reshape(-1)
    full = jnp.concatenate([pos, neg_indices]).reshape(-1, _PRED_STEPS)
    full = jnp.pad(full, ((0, 0), (0, _PSTEP_PAD - _PRED_STEPS)), mode="edge")
    g_idx = (
        jnp.arange(_B, dtype=jnp.int32)[:, None] * _T + full.reshape(-1)[None, :]
    ).reshape(-1)
    out = _sc_gather(x_flat, g_idx)
    out = out.reshape(_B, _NUM_NEG + 1, _TARGET_LEN, _PSTEP_PAD, _D)
    return out[:, :, :, :_PRED_STEPS, :]


# speedup vs baseline: 1.9631x; 1.1669x over previous
import jax
import jax.numpy as jnp
from jax import lax
from jax.experimental import pallas as pl
from jax.experimental.pallas import tpu as pltpu, tpu_sc as plsc
import functools

_T_SKIP = 4
_PRED_STEPS = 12
_NUM_NEG = 10
_B, _T, _D = 4, 512, 128
_TARGET_LEN = _T - _T_SKIP - _PRED_STEPS - 1
_NGROUP = _B * (_NUM_NEG + 1) * _TARGET_LEN
_N = _NGROUP * _PRED_STEPS

_NC, _NS = 2, 16
_NW = _NC * _NS
_RG = 684
_G = 18
_CROWS = _G * _PRED_STEPS
_NCHUNK = _RG // _G
_NBUF = 2
_NPAIR = _NCHUNK // _NBUF


def _gather_body(x_hbm, gidx_hbm, out_hbm, idx_v, buf0, buf1, g0, g1, s0, s1):
    bufs = (buf0, buf1)
    gsem = (g0, g1)
    ssem = (s0, s1)
    wid = lax.axis_index("s") * _NC + lax.axis_index("c")
    base_g = jnp.minimum(wid * _RG, _NGROUP - _RG)
    base_r = base_g * _PRED_STEPS
    pltpu.sync_copy(gidx_hbm.at[pl.ds(base_r, _RG * _PRED_STEPS)], idx_v)

    def pair(t, _):
        for b in range(_NBUF):
            c = t * _NBUF + b

            @pl.when(t > 0)
            def _():
                for _k in range(_G):
                    pltpu.make_async_copy(
                        bufs[b].at[pl.ds(0, _PRED_STEPS)], out_hbm.at[0], ssem[b]
                    ).wait()

            gcp = pltpu.async_copy(
                x_hbm.at[idx_v.at[pl.ds(c * _CROWS, _CROWS)]], bufs[b], gsem[b]
            )
            gcp.wait()
            goff = base_g + c * _G
            for k in range(_G):
                pltpu.async_copy(
                    bufs[b].at[pl.ds(k * _PRED_STEPS, _PRED_STEPS)],
                    out_hbm.at[goff + k],
                    ssem[b],
                )
        return ()

    lax.fori_loop(0, _NPAIR, pair, (), unroll=False)
    for b in range(_NBUF):
        for _k in range(_G):
            pltpu.make_async_copy(
                bufs[b].at[pl.ds(0, _PRED_STEPS)], out_hbm.at[0], ssem[b]
            ).wait()


@functools.partial(jax.jit)
def _sc_gather(x_flat, g_idx):
    mesh = plsc.VectorSubcoreMesh(
        core_axis_name="c", subcore_axis_name="s", num_cores=_NC, num_subcores=_NS
    )
    return pl.kernel(
        _gather_body,
        out_type=jax.ShapeDtypeStruct((_NGROUP, _PRED_STEPS, _D), jnp.float32),
        mesh=mesh,
        compiler_params=pltpu.CompilerParams(use_tc_tiling_on_sc=True),
        scratch_types=(
            [pltpu.VMEM((_RG * _PRED_STEPS,), jnp.int32)]
            + [pltpu.VMEM((_CROWS, _D), jnp.float32)] * _NBUF
            + [pltpu.SemaphoreType.DMA] * (2 * _NBUF)
        ),
    )(x_flat, g_idx)


def kernel(inputs, neg_indices):
    x_flat = inputs.reshape(_B * _T, _D)
    l = jnp.arange(_TARGET_LEN, dtype=jnp.int32)
    p = jnp.arange(_PRED_STEPS, dtype=jnp.int32)
    pos = (_T_SKIP + 1 + l[:, None] + p[None, :]).reshape(-1)
    full = jnp.concatenate([pos, neg_indices])
    g_idx = (
        jnp.arange(_B, dtype=jnp.int32)[:, None] * _T + full[None, :]
    ).reshape(-1)
    out = _sc_gather(x_flat, g_idx)
    return out.reshape(_B, _NUM_NEG + 1, _TARGET_LEN, _PRED_STEPS, _D)


# speedup vs baseline: 3.1963x; 1.6282x over previous
import jax
import jax.numpy as jnp
from jax import lax
from jax.experimental import pallas as pl
from jax.experimental.pallas import tpu as pltpu, tpu_sc as plsc
import functools

_T_SKIP = 4
_PRED_STEPS = 12
_NUM_NEG = 10
_B, _T, _D = 4, 512, 128
_TARGET_LEN = _T - _T_SKIP - _PRED_STEPS - 1
_NSLAB = (_NUM_NEG + 1) * _TARGET_LEN * _PRED_STEPS
_N = _NSLAB * _B

_NC, _NS = 2, 16
_NW = _NC * _NS
_R = 8160
_TAIL = _N - _NW * _R
_C = 408
_NCHUNK = _R // _C
_NBUF = 2
_NPAIR = _NCHUNK // _NBUF


def _gather_body(x_hbm, gidx_hbm, out_hbm, idx_v, tidx_v, buf0, buf1, g0, g1, s0, s1):
    bufs = (buf0, buf1)
    gsem = (g0, g1)
    ssem = (s0, s1)
    wid = lax.axis_index("s") * _NC + lax.axis_index("c")
    base = wid * _R
    pltpu.sync_copy(gidx_hbm.at[pl.ds(base, _R)], idx_v)

    def pair(t, _):
        for b in range(_NBUF):
            @pl.when(t > 0)
            def _():
                pltpu.make_async_copy(
                    bufs[b], out_hbm.at[pl.ds(0, _C)], ssem[b]
                ).wait()

            off = (t * _NBUF + b) * _C
            pltpu.async_copy(
                x_hbm.at[idx_v.at[pl.ds(off, _C)]], bufs[b], gsem[b]
            ).wait()
            pltpu.async_copy(bufs[b], out_hbm.at[pl.ds(base + off, _C)], ssem[b])
        return ()

    lax.fori_loop(0, _NPAIR, pair, (), unroll=False)
    for b in range(_NBUF):
        pltpu.make_async_copy(bufs[b], out_hbm.at[pl.ds(0, _C)], ssem[b]).wait()

    @pl.when(wid == _NW - 1)
    def _():
        tbase = _NW * _R
        pltpu.sync_copy(gidx_hbm.at[pl.ds(tbase, _TAIL)], tidx_v)
        pltpu.async_copy(
            x_hbm.at[tidx_v], bufs[0].at[pl.ds(0, _TAIL)], gsem[0]
        ).wait()
        pltpu.async_copy(
            bufs[0].at[pl.ds(0, _TAIL)], out_hbm.at[pl.ds(tbase, _TAIL)], ssem[0]
        ).wait()


@functools.partial(jax.jit)
def _sc_gather(x_tb, g_idx):
    mesh = plsc.VectorSubcoreMesh(
        core_axis_name="c", subcore_axis_name="s", num_cores=_NC, num_subcores=_NS
    )
    return pl.kernel(
        _gather_body,
        out_type=jax.ShapeDtypeStruct((_N, _D), jnp.float32),
        mesh=mesh,
        scratch_types=(
            [pltpu.VMEM((_R,), jnp.int32), pltpu.VMEM((_TAIL,), jnp.int32)]
            + [pltpu.VMEM((_C, _D), jnp.float32)] * _NBUF
            + [pltpu.SemaphoreType.DMA] * (2 * _NBUF)
        ),
    )(x_tb, g_idx)


def kernel(inputs, neg_indices):
    x_tb = jnp.transpose(inputs, (1, 0, 2)).reshape(_T * _B, _D)
    l = jnp.arange(_TARGET_LEN, dtype=jnp.int32)
    p = jnp.arange(_PRED_STEPS, dtype=jnp.int32)
    pos = (_T_SKIP + 1 + l[:, None] + p[None, :]).reshape(-1)
    full = jnp.concatenate([pos, neg_indices])
    g_idx = (
        full[:, None] * _B + jnp.arange(_B, dtype=jnp.int32)[None, :]
    ).reshape(-1)
    out = _sc_gather(x_tb, g_idx)
    out = out.reshape(_NUM_NEG + 1, _TARGET_LEN, _PRED_STEPS, _B, _D)
    return jnp.transpose(out, (3, 0, 1, 2, 4))
